# Initial kernel scaffold; baseline (speedup 1.0000x reference)
import jax
import jax.numpy as jnp
from jax.experimental import pallas as pl


def kernel(x, gate_W, gate_b, W1, b1, W2, b2):
    raise NotImplementedError("write your pallas kernel here")


# speedup vs baseline: 1.7597x; 1.7597x over previous
import jax
import jax.numpy as jnp
from jax.experimental import pallas as pl
from jax.experimental.pallas import tpu as pltpu

T = 4096
D_MODEL = 2048
D_FF = 2048
D_OUT = 2048
E = 8
TT = 512


def _moe_kernel(x_ref, gw_ref, gb_ref, w1_ref, b1_ref, w2_ref, b2_ref,
                out_ref, gate_scr):
    e = pl.program_id(1)

    x32 = x_ref[...]

    @pl.when(e == 0)
    def _():
        logits = jnp.dot(x32, gw_ref[...],
                         preferred_element_type=jnp.float32) + gb_ref[0]
        m = jnp.max(logits, axis=-1, keepdims=True)
        p = jnp.exp(logits - m)
        gate_scr[...] = p / jnp.sum(p, axis=-1, keepdims=True)

    xb = x32.astype(jnp.bfloat16)
    h = jnp.dot(xb, w1_ref[0], preferred_element_type=jnp.float32)
    h = jnp.maximum(h + b1_ref[0], 0.0).astype(jnp.bfloat16)
    y = jnp.dot(h, w2_ref[0], preferred_element_type=jnp.float32) + b2_ref[0]

    g = gate_scr[...]
    lane = jax.lax.broadcasted_iota(jnp.int32, g.shape, 1)
    ge = jnp.sum(jnp.where(lane == e, g, 0.0), axis=-1, keepdims=True)
    contrib = ge * y

    @pl.when(e == 0)
    def _():
        out_ref[...] = contrib

    @pl.when(e != 0)
    def _():
        out_ref[...] += contrib


def kernel(x, gate_W, gate_b, W1, b1, W2, b2):
    w1 = W1.astype(jnp.bfloat16)
    w2 = W2.astype(jnp.bfloat16)
    b1r = b1.reshape(E, 1, D_FF)
    b2r = b2.reshape(E, 1, D_OUT)
    gbr = gate_b.reshape(1, E)

    grid = (T // TT, E)
    return pl.pallas_call(
        _moe_kernel,
        grid=grid,
        in_specs=[
            pl.BlockSpec((TT, D_MODEL), lambda t, e: (t, 0)),
            pl.BlockSpec((D_MODEL, E), lambda t, e: (0, 0)),
            pl.BlockSpec((1, E), lambda t, e: (0, 0)),
            pl.BlockSpec((1, D_MODEL, D_FF), lambda t, e: (e, 0, 0)),
            pl.BlockSpec((1, 1, D_FF), lambda t, e: (e, 0, 0)),
            pl.BlockSpec((1, D_FF, D_OUT), lambda t, e: (e, 0, 0)),
            pl.BlockSpec((1, 1, D_OUT), lambda t, e: (e, 0, 0)),
        ],
        out_specs=pl.BlockSpec((TT, D_OUT), lambda t, e: (t, 0)),
        out_shape=jax.ShapeDtypeStruct((T, D_OUT), jnp.float32),
        scratch_shapes=[pltpu.VMEM((TT, E), jnp.float32)],
        compiler_params=pltpu.CompilerParams(
            dimension_semantics=("parallel", "arbitrary")),
    )(x, gate_W, gbr, w1, b1r, w2, b2r)
